# baseline (speedup 1.0000x reference)
import functools

import jax
import jax.numpy as jnp
from jax import lax
from jax.experimental import pallas as pl
from jax.experimental.pallas import tpu as pltpu
from jax.experimental.pallas import tpu_sc as plsc

N, E, DN, DE, L_TOT = 10000, 320000, 128, 16, 160000

_info = plsc.get_sparse_core_info()
NC, NS = _info.num_cores, _info.num_subcores
NW = NC * NS
BPW = L_TOT // NW
CB = 128
NFULL = BPW // CB
NTAIL = BPW - NFULL * CB
NBUF = 3
NGRP = NFULL // NBUF

ECT = E // 128
ECT_BASE = ECT // NW
NBUFD = 6
NGRPD = (ECT_BASE + 1 + 2 * NBUFD - 1) // NBUFD + 1
OCT = L_TOT // 128
OCT_BASE = OCT // NW
NGRPE = (OCT_BASE + 1 + 2 * NBUF - 1) // NBUF + 1


@functools.partial(
    pl.kernel,
    mesh=plsc.VectorSubcoreMesh(core_axis_name="c", subcore_axis_name="s"),
    out_type=jax.ShapeDtypeStruct((E, DE), jnp.float32),
    scratch_types=[
        pltpu.VMEM((NBUFD, 2, 8, 128), jnp.float32),
        pltpu.VMEM((NBUFD, 128, 17), jnp.float32),
        pltpu.SemaphoreType.DMA((NBUFD,)),
        pltpu.SemaphoreType.DMA((NBUFD,)),
    ],
    compiler_params=pltpu.CompilerParams(use_tc_tiling_on_sc=False,
                                         needs_layout_passes=False),
)
def _edge_detile_sc(e4, erow, tbuf, rbuf, gsem, ssem):
    wid = lax.axis_index("s") * NC + lax.axis_index("c")
    lo = (wid * ECT) // NW
    ntiles = ((wid + 1) * ECT) // NW - lo
    lane = lax.iota(jnp.int32, 16)
    rt_c = [jnp.full((16,), r // 8, jnp.int32) for r in range(16)]
    row_c = [jnp.full((16,), r % 8, jnp.int32) for r in range(16)]
    r_c = [jnp.full((16,), r, jnp.int32) for r in range(16)]

    def fire(t, b):
        ct = lo + t
        pltpu.make_async_copy(e4.at[0, ct], tbuf.at[b, 0], gsem.at[b]).start()
        pltpu.make_async_copy(e4.at[1, ct], tbuf.at[b, 1], gsem.at[b]).start()

    def wait_in(b):
        for rt in range(2):
            pltpu.make_async_copy(e4.at[rt, 0], tbuf.at[b, rt],
                                  gsem.at[b]).wait()

    def transpose(b):
        def tcol(c0, carry):
            colv = lane + c0 * 16
            for r in range(16):
                v = plsc.load_gather(tbuf.at[b], [rt_c[r], row_c[r], colv])
                plsc.store_scatter(rbuf.at[b], [colv, r_c[r]], v)
            return carry
        lax.fori_loop(0, 8, tcol, 0)

    def store(t, b):
        pltpu.make_async_copy(
            rbuf.at[b, :, pl.ds(0, 16)],
            erow.at[pl.ds((lo + t) * 128, 128)], ssem.at[b]).start()

    def wait_out(b):
        pltpu.make_async_copy(
            rbuf.at[b, :, pl.ds(0, 16)],
            erow.at[pl.ds(0, 128)], ssem.at[b]).wait()

    for b in range(NBUFD):
        fire(b, b)
    for b in range(NBUFD):
        wait_in(b)
        transpose(b)
        store(b, b)
        fire(b + NBUFD, b)

    def grp(g, carry):
        for b in range(NBUFD):
            t = g * NBUFD + b

            @pl.when(t < ntiles + NBUFD)
            def _():
                wait_out(b)

            @pl.when(t < ntiles)
            def _():
                wait_in(b)
                transpose(b)
                store(t, b)

            @pl.when(t + NBUFD < ntiles)
            def _():
                fire(t + NBUFD, b)
        return carry

    lax.fori_loop(1, NGRPD, grp, 0)


@functools.partial(
    pl.kernel,
    mesh=plsc.VectorSubcoreMesh(core_axis_name="c", subcore_axis_name="s"),
    out_type=(
        [jax.ShapeDtypeStruct((L_TOT, DN), jnp.float32)] * 3
        + [jax.ShapeDtypeStruct((2, OCT, 8, 128), jnp.float32)] * 3
    ),
    scratch_types=[
        pltpu.VMEM((OCT_BASE * CB + CB,), jnp.int32),
        pltpu.VMEM((NBUF, CB, DN), jnp.float32),
        pltpu.VMEM((NBUF, CB, DE), jnp.float32),
        pltpu.VMEM((NBUF, 2, 8, 133), jnp.float32),
        pltpu.VMEM((NTAIL, DN), jnp.float32),
        pltpu.SemaphoreType.DMA((NBUF,)),
        pltpu.SemaphoreType.DMA((NBUF,)),
        pltpu.SemaphoreType.DMA,
    ],
    compiler_params=pltpu.CompilerParams(use_tc_tiling_on_sc=False,
                                         needs_layout_passes=False),
)
def _tree_gather_sc(node_hbm, erow, n1, n2, n3, e1, e2, e3,
                    on1, on2, on3, oe1, oe2, oe3,
                    idx_v, nrow_v, grow_v, obuf_v, ntail_v, gsem, ssem, tsem):
    wid = lax.axis_index("s") * NC + lax.axis_index("c")
    lane = lax.iota(jnp.int32, 16)
    k8a = lane // 8
    k8b = lane % 8

    def run_node(idx_hbm, out):
        base = wid * BPW
        pltpu.sync_copy(idx_hbm.at[pl.ds(base, BPW)],
                        idx_v.at[pl.ds(0, BPW)])

        tail_g = pltpu.make_async_copy(
            node_hbm.at[idx_v.at[pl.ds(NFULL * CB, NTAIL)]], ntail_v, tsem)
        tail_g.start()

        def fire(j, b):
            pltpu.make_async_copy(
                node_hbm.at[idx_v.at[pl.ds(j * CB, CB)]], nrow_v.at[b],
                gsem.at[b]).start()

        def wait_gather(b):
            pltpu.make_async_copy(
                node_hbm.at[idx_v.at[pl.ds(0, CB)]], nrow_v.at[b],
                gsem.at[b]).wait()

        def store(j, b):
            pltpu.make_async_copy(
                nrow_v.at[b], out.at[pl.ds(base + j * CB, CB)], ssem.at[b]
            ).start()

        def wait_store(b):
            pltpu.make_async_copy(
                nrow_v.at[b], out.at[pl.ds(base, CB)], ssem.at[b]).wait()

        for b in range(NBUF):
            fire(b, b)

        def grp(g, carry):
            for b in range(NBUF):
                j = g * NBUF + b
                wait_gather(b)
                store(j, b)
            for b in range(NBUF):
                wait_store(b)
                fire(g * NBUF + b + NBUF, b)
            return carry

        lax.fori_loop(0, NGRP - 1, grp, 0)

        g = NGRP - 1
        for b in range(NBUF):
            wait_gather(b)
            store(g * NBUF + b, b)
        tail_g.wait()
        tail_s = pltpu.make_async_copy(
            ntail_v, out.at[pl.ds(base + NFULL * CB, NTAIL)], tsem)
        tail_s.start()
        for b in range(NBUF):
            wait_store(b)
        tail_s.wait()

    def run_edge(idx_hbm, o4):
        lo = (wid * OCT) // NW
        ntiles = ((wid + 1) * OCT) // NW - lo

        pltpu.sync_copy(idx_hbm.at[pl.ds(lo * CB, OCT_BASE * CB)],
                        idx_v.at[pl.ds(0, OCT_BASE * CB)])

        @pl.when(ntiles > OCT_BASE)
        def _():
            pltpu.sync_copy(
                idx_hbm.at[pl.ds((lo + OCT_BASE) * CB, CB)],
                idx_v.at[pl.ds(OCT_BASE * CB, CB)])

        def fire(t, b):
            pltpu.make_async_copy(
                erow.at[idx_v.at[pl.ds(t * CB, CB)]], grow_v.at[b],
                gsem.at[b]).start()

        def wait_gather(b):
            pltpu.make_async_copy(
                erow.at[idx_v.at[pl.ds(0, CB)]], grow_v.at[b],
                gsem.at[b]).wait()

        def transpose(b):
            def tcol(ci, carry):
                for u in range(16):
                    cs = jnp.full((16,), ci * 16 + u, jnp.int32)
                    v = plsc.load_gather(grow_v.at[b], [cs, lane])
                    plsc.store_scatter(obuf_v.at[b], [k8a, k8b, cs], v)
                return carry
            lax.fori_loop(0, 8, tcol, 0)

        def store(t, b):
            ct = lo + t
            for rt in range(2):
                pltpu.make_async_copy(
                    obuf_v.at[b, rt, :, pl.ds(0, 128)],
                    o4.at[rt, ct], ssem.at[b]).start()

        def wait_out(b):
            for rt in range(2):
                pltpu.make_async_copy(
                    obuf_v.at[b, rt, :, pl.ds(0, 128)],
                    o4.at[rt, 0], ssem.at[b]).wait()

        for b in range(NBUF):
            fire(b, b)
        for b in range(NBUF):
            wait_gather(b)
            transpose(b)
            store(b, b)
            fire(b + NBUF, b)

        def grp(g, carry):
            for b in range(NBUF):
                t = g * NBUF + b

                @pl.when(t < ntiles + NBUF)
                def _():
                    wait_out(b)

                @pl.when(t < ntiles)
                def _():
                    wait_gather(b)
                    transpose(b)
                    store(t, b)

                @pl.when(t + NBUF < ntiles)
                def _():
                    fire(t + NBUF, b)
            return carry

        lax.fori_loop(1, NGRPE, grp, 0)

    run_node(n1, on1)
    run_node(n2, on2)
    run_node(n3, on3)
    run_edge(e1, oe1)
    run_edge(e2, oe2)
    run_edge(e3, oe3)


def kernel(node_feat, edge_feat, n_img_1, n_img_2, n_img_3,
           e_img_1, e_img_2, e_img_3):
    e4 = (edge_feat.T.reshape(2, 8, ECT, 128).transpose(0, 2, 1, 3))
    erow = _edge_detile_sc(e4)
    on1, on2, on3, oe1_4, oe2_4, oe3_4 = _tree_gather_sc(
        node_feat, erow, n_img_1, n_img_2, n_img_3,
        e_img_1, e_img_2, e_img_3)

    def untile(o4):
        return (o4.transpose(0, 2, 1, 3).reshape(16, L_TOT).T)

    return (node_feat, on1, on2, on3,
            edge_feat, untile(oe1_4), untile(oe2_4), untile(oe3_4))


# speedup vs baseline: 1.2216x; 1.2216x over previous
import functools

import jax
import jax.numpy as jnp
from jax import lax
from jax.experimental import pallas as pl
from jax.experimental.pallas import tpu as pltpu
from jax.experimental.pallas import tpu_sc as plsc

N, E, DN, DE, L_TOT = 10000, 320000, 128, 16, 160000

_info = plsc.get_sparse_core_info()
NC, NS = _info.num_cores, _info.num_subcores
NW = NC * NS
BPW = L_TOT // NW
CB = 128
NFULL = BPW // CB
NTAIL = BPW - NFULL * CB
NBUF = 3
NGRP = NFULL // NBUF
CBN = 104
NFULLN = BPW // CBN
NTAILN = BPW - NFULLN * CBN
NGRPN = NFULLN // NBUF

ECT = E // 128
ECT_BASE = ECT // NW
NBUFD = 6
NGRPD = (ECT_BASE + 1 + 2 * NBUFD - 1) // NBUFD + 1
OCT = L_TOT // 128
OCT_BASE = OCT // NW
NGRPE = (OCT_BASE + 1 + 2 * NBUF - 1) // NBUF + 1


@functools.partial(
    pl.kernel,
    mesh=plsc.VectorSubcoreMesh(core_axis_name="c", subcore_axis_name="s"),
    out_type=jax.ShapeDtypeStruct((E, DE), jnp.float32),
    scratch_types=[
        pltpu.VMEM((NBUFD, 2, 8, 128), jnp.float32),
        pltpu.VMEM((NBUFD, 128, 17), jnp.float32),
        pltpu.SemaphoreType.DMA((NBUFD,)),
        pltpu.SemaphoreType.DMA((NBUFD,)),
    ],
    compiler_params=pltpu.CompilerParams(use_tc_tiling_on_sc=False,
                                         needs_layout_passes=False),
)
def _edge_detile_sc(e4, erow, tbuf, rbuf, gsem, ssem):
    wid = lax.axis_index("s") * NC + lax.axis_index("c")
    lo = (wid * ECT) // NW
    ntiles = ((wid + 1) * ECT) // NW - lo
    lane = lax.iota(jnp.int32, 16)
    rt_c = [jnp.full((16,), r // 8, jnp.int32) for r in range(16)]
    row_c = [jnp.full((16,), r % 8, jnp.int32) for r in range(16)]
    r_c = [jnp.full((16,), r, jnp.int32) for r in range(16)]

    def fire(t, b):
        ct = lo + t
        pltpu.make_async_copy(e4.at[0, ct], tbuf.at[b, 0], gsem.at[b]).start()
        pltpu.make_async_copy(e4.at[1, ct], tbuf.at[b, 1], gsem.at[b]).start()

    def wait_in(b):
        for rt in range(2):
            pltpu.make_async_copy(e4.at[rt, 0], tbuf.at[b, rt],
                                  gsem.at[b]).wait()

    def transpose(b):
        def tcol(c0, carry):
            colv = lane + c0 * 16
            for r in range(16):
                v = plsc.load_gather(tbuf.at[b], [rt_c[r], row_c[r], colv])
                plsc.store_scatter(rbuf.at[b], [colv, r_c[r]], v)
            return carry
        lax.fori_loop(0, 8, tcol, 0)

    def store(t, b):
        pltpu.make_async_copy(
            rbuf.at[b, :, pl.ds(0, 16)],
            erow.at[pl.ds((lo + t) * 128, 128)], ssem.at[b]).start()

    def wait_out(b):
        pltpu.make_async_copy(
            rbuf.at[b, :, pl.ds(0, 16)],
            erow.at[pl.ds(0, 128)], ssem.at[b]).wait()

    for b in range(NBUFD):
        fire(b, b)
    for b in range(NBUFD):
        wait_in(b)
        transpose(b)
        store(b, b)
        fire(b + NBUFD, b)

    def grp(g, carry):
        for b in range(NBUFD):
            t = g * NBUFD + b

            @pl.when(t < ntiles + NBUFD)
            def _():
                wait_out(b)

            @pl.when(t < ntiles)
            def _():
                wait_in(b)
                transpose(b)
                store(t, b)

            @pl.when(t + NBUFD < ntiles)
            def _():
                fire(t + NBUFD, b)
        return carry

    lax.fori_loop(1, NGRPD, grp, 0)


@functools.partial(
    pl.kernel,
    mesh=plsc.VectorSubcoreMesh(core_axis_name="c", subcore_axis_name="s"),
    out_type=[jax.ShapeDtypeStruct((L_TOT, DN), jnp.float32)] * 3,
    scratch_types=[
        pltpu.VMEM((BPW,), jnp.int32),
        pltpu.VMEM((CBN, DN), jnp.float32),
        pltpu.VMEM((CBN, DN), jnp.float32),
        pltpu.VMEM((CBN, DN), jnp.float32),
        pltpu.VMEM((NTAILN, DN), jnp.float32),
        pltpu.VMEM_SHARED((N, DN), jnp.float32),
        pltpu.SemaphoreType.DMA((NBUF,)),
        pltpu.SemaphoreType.DMA((NBUF,)),
        pltpu.SemaphoreType.DMA,
    ],
    compiler_params=pltpu.CompilerParams(use_tc_tiling_on_sc=False,
                                         needs_layout_passes=False,
                                         internal_scratch_in_bytes=0),
)
def _node_gather_sc(node_hbm, n1, n2, n3, on1, on2, on3,
                    idx_v, nrow0, nrow1, nrow2, ntail_v, nspm,
                    gsem, ssem, tsem):
    nrow = (nrow0, nrow1, nrow2)
    sid = lax.axis_index("s")
    wid = sid * NC + lax.axis_index("c")

    rows_per_sub = N // NS
    pltpu.sync_copy(node_hbm.at[pl.ds(sid * rows_per_sub, rows_per_sub)],
                    nspm.at[pl.ds(sid * rows_per_sub, rows_per_sub)])
    plsc.subcore_barrier()

    def run_node(idx_hbm, out):
        base = wid * BPW
        pltpu.sync_copy(idx_hbm.at[pl.ds(base, BPW)],
                        idx_v.at[pl.ds(0, BPW)])

        tail_g = pltpu.make_async_copy(
            nspm.at[idx_v.at[pl.ds(NFULLN * CBN, NTAILN)]], ntail_v, tsem)
        tail_g.start()

        def fire(j, b):
            pltpu.make_async_copy(
                nspm.at[idx_v.at[pl.ds(j * CBN, CBN)]], nrow[b],
                gsem.at[b]).start()

        def wait_gather(b):
            pltpu.make_async_copy(
                nspm.at[idx_v.at[pl.ds(0, CBN)]], nrow[b],
                gsem.at[b]).wait()

        def store(j, b):
            pltpu.make_async_copy(
                nrow[b], out.at[pl.ds(base + j * CBN, CBN)], ssem.at[b]
            ).start()

        def wait_store(b):
            pltpu.make_async_copy(
                nrow[b], out.at[pl.ds(base, CBN)], ssem.at[b]).wait()

        for b in range(NBUF):
            fire(b, b)

        def grp(g, carry):
            for b in range(NBUF):
                j = g * NBUF + b
                wait_gather(b)
                store(j, b)
            for b in range(NBUF):
                wait_store(b)
                fire(g * NBUF + b + NBUF, b)
            return carry

        lax.fori_loop(0, NGRPN - 1, grp, 0)

        g = NGRPN - 1
        for b in range(NBUF):
            wait_gather(b)
            store(g * NBUF + b, b)
        tail_g.wait()
        tail_s = pltpu.make_async_copy(
            ntail_v, out.at[pl.ds(base + NFULLN * CBN, NTAILN)], tsem)
        tail_s.start()
        for b in range(NBUF):
            wait_store(b)
        tail_s.wait()

    run_node(n1, on1)
    run_node(n2, on2)
    run_node(n3, on3)


@functools.partial(
    pl.kernel,
    mesh=plsc.VectorSubcoreMesh(core_axis_name="c", subcore_axis_name="s"),
    out_type=[jax.ShapeDtypeStruct((2, OCT, 8, 128), jnp.float32)] * 3,
    scratch_types=[
        pltpu.VMEM((OCT_BASE * CB + CB,), jnp.int32),
        pltpu.VMEM((NBUF, CB, DE), jnp.float32),
        pltpu.VMEM((NBUF, 2, 8, 133), jnp.float32),
        pltpu.SemaphoreType.DMA((NBUF,)),
        pltpu.SemaphoreType.DMA((NBUF,)),
    ],
    compiler_params=pltpu.CompilerParams(use_tc_tiling_on_sc=False,
                                         needs_layout_passes=False),
)
def _edge_gather_sc(erow, e1, e2, e3, oe1, oe2, oe3,
                    idx_v, grow_v, obuf_v, gsem, ssem):
    wid = lax.axis_index("s") * NC + lax.axis_index("c")
    lane = lax.iota(jnp.int32, 16)
    k8a = lane // 8
    k8b = lane % 8

    def run_edge(idx_hbm, o4):
        lo = (wid * OCT) // NW
        ntiles = ((wid + 1) * OCT) // NW - lo

        pltpu.sync_copy(idx_hbm.at[pl.ds(lo * CB, OCT_BASE * CB)],
                        idx_v.at[pl.ds(0, OCT_BASE * CB)])

        @pl.when(ntiles > OCT_BASE)
        def _():
            pltpu.sync_copy(
                idx_hbm.at[pl.ds((lo + OCT_BASE) * CB, CB)],
                idx_v.at[pl.ds(OCT_BASE * CB, CB)])

        def fire(t, b):
            pltpu.make_async_copy(
                erow.at[idx_v.at[pl.ds(t * CB, CB)]], grow_v.at[b],
                gsem.at[b]).start()

        def wait_gather(b):
            pltpu.make_async_copy(
                erow.at[idx_v.at[pl.ds(0, CB)]], grow_v.at[b],
                gsem.at[b]).wait()

        def transpose(b):
            def tcol(ci, carry):
                for u in range(16):
                    cs = jnp.full((16,), ci * 16 + u, jnp.int32)
                    v = plsc.load_gather(grow_v.at[b], [cs, lane])
                    plsc.store_scatter(obuf_v.at[b], [k8a, k8b, cs], v)
                return carry
            lax.fori_loop(0, 8, tcol, 0)

        def store(t, b):
            ct = lo + t
            for rt in range(2):
                pltpu.make_async_copy(
                    obuf_v.at[b, rt, :, pl.ds(0, 128)],
                    o4.at[rt, ct], ssem.at[b]).start()

        def wait_out(b):
            for rt in range(2):
                pltpu.make_async_copy(
                    obuf_v.at[b, rt, :, pl.ds(0, 128)],
                    o4.at[rt, 0], ssem.at[b]).wait()

        for b in range(NBUF):
            fire(b, b)
        for b in range(NBUF):
            wait_gather(b)
            transpose(b)
            store(b, b)
            fire(b + NBUF, b)

        def grp(g, carry):
            for b in range(NBUF):
                t = g * NBUF + b

                @pl.when(t < ntiles + NBUF)
                def _():
                    wait_out(b)

                @pl.when(t < ntiles)
                def _():
                    wait_gather(b)
                    transpose(b)
                    store(t, b)

                @pl.when(t + NBUF < ntiles)
                def _():
                    fire(t + NBUF, b)
            return carry

        lax.fori_loop(1, NGRPE, grp, 0)

    run_edge(e1, oe1)
    run_edge(e2, oe2)
    run_edge(e3, oe3)


def kernel(node_feat, edge_feat, n_img_1, n_img_2, n_img_3,
           e_img_1, e_img_2, e_img_3):
    e4 = (edge_feat.T.reshape(2, 8, ECT, 128).transpose(0, 2, 1, 3))
    erow = _edge_detile_sc(e4)
    on1, on2, on3 = _node_gather_sc(node_feat, n_img_1, n_img_2, n_img_3)
    oe1_4, oe2_4, oe3_4 = _edge_gather_sc(erow, e_img_1, e_img_2, e_img_3)

    def untile(o4):
        return (o4.transpose(0, 2, 1, 3).reshape(16, L_TOT).T)

    return (node_feat, on1, on2, on3,
            edge_feat, untile(oe1_4), untile(oe2_4), untile(oe3_4))


# speedup vs baseline: 1.3655x; 1.1178x over previous
import functools

import jax
import jax.numpy as jnp
from jax import lax
from jax.experimental import pallas as pl
from jax.experimental.pallas import tpu as pltpu
from jax.experimental.pallas import tpu_sc as plsc

N, E, DN, DE, L_TOT = 10000, 320000, 128, 16, 160000

_info = plsc.get_sparse_core_info()
NC, NS = _info.num_cores, _info.num_subcores
NW = NC * NS
BPW = L_TOT // NW
CB = 128
NFULL = BPW // CB
NTAIL = BPW - NFULL * CB
NBUF = 3
NGRP = NFULL // NBUF
CBN = 104
NFULLN = BPW // CBN
NTAILN = BPW - NFULLN * CBN
NGRPN = NFULLN // NBUF

ECT = E // 128
ECT_BASE = ECT // NW
NBUFD = 6
NGRPD = (ECT_BASE + 1 + 2 * NBUFD - 1) // NBUFD + 1
OCT = L_TOT // 128
OCT_BASE = OCT // NW
NGRPE = (OCT_BASE + 1 + 2 * NBUF - 1) // NBUF + 1


@functools.partial(
    pl.kernel,
    mesh=plsc.VectorSubcoreMesh(core_axis_name="c", subcore_axis_name="s"),
    out_type=[jax.ShapeDtypeStruct((L_TOT, DN), jnp.float32)] * 3,
    scratch_types=[
        pltpu.VMEM((BPW,), jnp.int32),
        pltpu.VMEM((CBN, DN), jnp.float32),
        pltpu.VMEM((CBN, DN), jnp.float32),
        pltpu.VMEM((CBN, DN), jnp.float32),
        pltpu.VMEM((NTAILN, DN), jnp.float32),
        pltpu.VMEM_SHARED((N, DN), jnp.float32),
        pltpu.SemaphoreType.DMA((NBUF,)),
        pltpu.SemaphoreType.DMA((NBUF,)),
        pltpu.SemaphoreType.DMA,
    ],
    compiler_params=pltpu.CompilerParams(use_tc_tiling_on_sc=False,
                                         needs_layout_passes=False,
                                         internal_scratch_in_bytes=0),
)
def _node_gather_sc(node_hbm, n1, n2, n3, on1, on2, on3,
                    idx_v, nrow0, nrow1, nrow2, ntail_v, nspm,
                    gsem, ssem, tsem):
    nrow = (nrow0, nrow1, nrow2)
    sid = lax.axis_index("s")
    wid = sid * NC + lax.axis_index("c")

    rows_per_sub = N // NS
    pltpu.sync_copy(node_hbm.at[pl.ds(sid * rows_per_sub, rows_per_sub)],
                    nspm.at[pl.ds(sid * rows_per_sub, rows_per_sub)])
    plsc.subcore_barrier()

    def run_node(idx_hbm, out):
        base = wid * BPW
        pltpu.sync_copy(idx_hbm.at[pl.ds(base, BPW)],
                        idx_v.at[pl.ds(0, BPW)])

        tail_g = pltpu.make_async_copy(
            nspm.at[idx_v.at[pl.ds(NFULLN * CBN, NTAILN)]], ntail_v, tsem)
        tail_g.start()

        def fire(j, b):
            pltpu.make_async_copy(
                nspm.at[idx_v.at[pl.ds(j * CBN, CBN)]], nrow[b],
                gsem.at[b]).start()

        def wait_gather(b):
            pltpu.make_async_copy(
                nspm.at[idx_v.at[pl.ds(0, CBN)]], nrow[b],
                gsem.at[b]).wait()

        def store(j, b):
            pltpu.make_async_copy(
                nrow[b], out.at[pl.ds(base + j * CBN, CBN)], ssem.at[b]
            ).start()

        def wait_store(b):
            pltpu.make_async_copy(
                nrow[b], out.at[pl.ds(base, CBN)], ssem.at[b]).wait()

        for b in range(NBUF):
            fire(b, b)

        def grp(g, carry):
            for b in range(NBUF):
                j = g * NBUF + b
                wait_gather(b)
                store(j, b)
            for b in range(NBUF):
                wait_store(b)
                fire(g * NBUF + b + NBUF, b)
            return carry

        lax.fori_loop(0, NGRPN - 1, grp, 0)

        g = NGRPN - 1
        for b in range(NBUF):
            wait_gather(b)
            store(g * NBUF + b, b)
        tail_g.wait()
        tail_s = pltpu.make_async_copy(
            ntail_v, out.at[pl.ds(base + NFULLN * CBN, NTAILN)], tsem)
        tail_s.start()
        for b in range(NBUF):
            wait_store(b)
        tail_s.wait()

    run_node(n1, on1)
    run_node(n2, on2)
    run_node(n3, on3)


@functools.partial(
    pl.kernel,
    mesh=plsc.VectorSubcoreMesh(core_axis_name="c", subcore_axis_name="s"),
    out_type=[jax.ShapeDtypeStruct((2, OCT, 8, 128), jnp.float32)] * 3,
    scratch_types=[
        pltpu.VMEM((OCT_BASE * CB + CB,), jnp.int32),
        pltpu.VMEM((NBUF, CB, DE), jnp.float32),
        pltpu.VMEM((NBUF, 2, 8, 133), jnp.float32),
        pltpu.SemaphoreType.DMA((NBUF,)),
        pltpu.SemaphoreType.DMA((NBUF,)),
    ],
    compiler_params=pltpu.CompilerParams(use_tc_tiling_on_sc=False,
                                         needs_layout_passes=False),
)
def _edge_gather_sc(erow, e1, e2, e3, dep, oe1, oe2, oe3,
                    idx_v, grow_v, obuf_v, gsem, ssem):
    del dep
    wid = lax.axis_index("s") * NC + lax.axis_index("c")
    lane = lax.iota(jnp.int32, 16)
    k8a = lane // 8
    k8b = lane % 8

    def run_edge(idx_hbm, o4):
        lo = (wid * OCT) // NW
        ntiles = ((wid + 1) * OCT) // NW - lo

        pltpu.sync_copy(idx_hbm.at[pl.ds(lo * CB, OCT_BASE * CB)],
                        idx_v.at[pl.ds(0, OCT_BASE * CB)])

        @pl.when(ntiles > OCT_BASE)
        def _():
            pltpu.sync_copy(
                idx_hbm.at[pl.ds((lo + OCT_BASE) * CB, CB)],
                idx_v.at[pl.ds(OCT_BASE * CB, CB)])

        def fire(t, b):
            pltpu.make_async_copy(
                erow.at[idx_v.at[pl.ds(t * CB, CB)]], grow_v.at[b],
                gsem.at[b]).start()

        def wait_gather(b):
            pltpu.make_async_copy(
                erow.at[idx_v.at[pl.ds(0, CB)]], grow_v.at[b],
                gsem.at[b]).wait()

        def transpose(b):
            def tcol(ci, carry):
                for u in range(16):
                    cs = jnp.full((16,), ci * 16 + u, jnp.int32)
                    v = plsc.load_gather(grow_v.at[b], [cs, lane])
                    plsc.store_scatter(obuf_v.at[b], [k8a, k8b, cs], v)
                return carry
            lax.fori_loop(0, 8, tcol, 0)

        def store(t, b):
            ct = lo + t
            for rt in range(2):
                pltpu.make_async_copy(
                    obuf_v.at[b, rt, :, pl.ds(0, 128)],
                    o4.at[rt, ct], ssem.at[b]).start()

        def wait_out(b):
            for rt in range(2):
                pltpu.make_async_copy(
                    obuf_v.at[b, rt, :, pl.ds(0, 128)],
                    o4.at[rt, 0], ssem.at[b]).wait()

        for b in range(NBUF):
            fire(b, b)
        for b in range(NBUF):
            wait_gather(b)
            transpose(b)
            store(b, b)
            fire(b + NBUF, b)

        def grp(g, carry):
            for b in range(NBUF):
                t = g * NBUF + b

                @pl.when(t < ntiles + NBUF)
                def _():
                    wait_out(b)

                @pl.when(t < ntiles)
                def _():
                    wait_gather(b)
                    transpose(b)
                    store(t, b)

                @pl.when(t + NBUF < ntiles)
                def _():
                    fire(t + NBUF, b)
            return carry

        lax.fori_loop(1, NGRPE, grp, 0)

    run_edge(e1, oe1)
    run_edge(e2, oe2)
    run_edge(e3, oe3)


def kernel(node_feat, edge_feat, n_img_1, n_img_2, n_img_3,
           e_img_1, e_img_2, e_img_3):
    on1, on2, on3 = _node_gather_sc(node_feat, n_img_1, n_img_2, n_img_3)
    oe1_4, oe2_4, oe3_4 = _edge_gather_sc(edge_feat, e_img_1, e_img_2,
                                          e_img_3, on1)

    def untile(o4):
        return (o4.transpose(0, 2, 1, 3).reshape(16, L_TOT).T)

    return (node_feat, on1, on2, on3,
            edge_feat, untile(oe1_4), untile(oe2_4), untile(oe3_4))
